# baseline (speedup 1.0000x reference)
import functools

import jax
import jax.numpy as jnp
from jax.experimental import pallas as pl
from jax.experimental.pallas import tpu as pltpu

K = 16
Q = 8


def _knn_body(q_ref, k_ref, o_ref):
    queries = q_ref[0]
    keys = k_ref[0]
    n = keys.shape[0]

    keys_tt = keys.T
    diff = jnp.abs(queries[:, :, None] - keys_tt[None, :, :])
    dist = jnp.sum(diff, axis=1)

    col = jax.lax.broadcasted_iota(jnp.int32, dist.shape, 1)

    idx_cols = []
    for _ in range(K + 1):
        mn = jnp.min(dist, axis=1, keepdims=True)
        eq = dist == mn
        idxv = jnp.min(jnp.where(eq, col, n), axis=1)
        idx_cols.append(idxv)
        dist = jnp.where(col == idxv[:, None], jnp.inf, dist)
    idx = jnp.stack(idx_cols, axis=1)

    keys_t = keys.T
    row = jax.lax.broadcasted_iota(jnp.int32, (n, K + 1), 0)
    for q in range(idx.shape[0]):
        oh = (row == idx[q:q + 1, :]).astype(keys.dtype)
        o_ref[0, q] = jnp.dot(keys_t, oh,
                              preferred_element_type=jnp.float32,
                              precision=jax.lax.Precision.HIGHEST)


def kernel(inputs):
    B, N, F = inputs.shape
    grid = (B, N // Q)
    out = pl.pallas_call(
        _knn_body,
        grid=grid,
        in_specs=[
            pl.BlockSpec((1, Q, F), lambda b, nb: (b, nb, 0)),
            pl.BlockSpec((1, N, F), lambda b, nb: (b, 0, 0)),
        ],
        out_specs=pl.BlockSpec((1, Q, F, K + 1), lambda b, nb: (b, nb, 0, 0)),
        out_shape=jax.ShapeDtypeStruct((B, N, F, K + 1), jnp.float32),
        compiler_params=pltpu.CompilerParams(
            dimension_semantics=("parallel", "arbitrary"),
        ),
    )(inputs, inputs)
    return out


# speedup vs baseline: 1.0760x; 1.0760x over previous
import functools

import jax
import jax.numpy as jnp
from jax.experimental import pallas as pl
from jax.experimental.pallas import tpu as pltpu

K = 16
Q = 8


def _knn_body(q_ref, kt_ref, k_ref, o_ref):
    keys_t = kt_ref[0]
    keys = k_ref[0]
    n = keys.shape[0]
    queries_t = q_ref[0].T

    rows = []
    for q in range(Q):
        dq = jnp.abs(queries_t[:, q:q + 1] - keys_t)
        rows.append(jnp.sum(dq, axis=0, keepdims=True))
    dist = jnp.concatenate(rows, axis=0)

    col = jax.lax.broadcasted_iota(jnp.int32, dist.shape, 1)

    for j in range(K + 1):
        mn = jnp.min(dist, axis=1, keepdims=True)
        eq = dist == mn
        idxv = jnp.min(jnp.where(eq, col, n), axis=1)
        sel = col == idxv[:, None]
        dist = jnp.where(sel, jnp.inf, dist)
        nb = jnp.dot(sel.astype(jnp.float32), keys,
                     preferred_element_type=jnp.float32,
                     precision=jax.lax.Precision.HIGHEST)
        o_ref[0, :, :, j] = nb


def kernel(inputs):
    B, N, F = inputs.shape
    inputs_t = jnp.transpose(inputs, (0, 2, 1))
    grid = (B, N // Q)
    out = pl.pallas_call(
        _knn_body,
        grid=grid,
        in_specs=[
            pl.BlockSpec((1, Q, F), lambda b, nb: (b, nb, 0)),
            pl.BlockSpec((1, F, N), lambda b, nb: (b, 0, 0)),
            pl.BlockSpec((1, N, F), lambda b, nb: (b, 0, 0)),
        ],
        out_specs=pl.BlockSpec((1, Q, F, K + 1), lambda b, nb: (b, nb, 0, 0)),
        out_shape=jax.ShapeDtypeStruct((B, N, F, K + 1), jnp.float32),
        compiler_params=pltpu.CompilerParams(
            dimension_semantics=("parallel", "arbitrary"),
        ),
    )(inputs, inputs_t, inputs)
    return out


# speedup vs baseline: 1.3836x; 1.2859x over previous
import functools

import jax
import jax.numpy as jnp
from jax.experimental import pallas as pl
from jax.experimental.pallas import tpu as pltpu

K = 16
Q = 8


def _knn_body(q_ref, kt_ref, k_ref, o_ref):
    keys_t = kt_ref[0]
    keys = k_ref[0]
    n = keys.shape[0]
    queries_t = q_ref[0].T

    rows = []
    for q in range(Q):
        dq = jnp.abs(queries_t[:, q:q + 1] - keys_t)
        rows.append(jnp.sum(dq, axis=0, keepdims=True))
    dist = jnp.concatenate(rows, axis=0)

    col = jax.lax.broadcasted_iota(jnp.int32, dist.shape, 1)

    for j in range(K + 1):
        mn = jnp.min(dist, axis=1, keepdims=True)
        eq = dist == mn
        idxv = jnp.min(jnp.where(eq, col, n), axis=1)
        sel = col == idxv[:, None]
        dist = jnp.where(sel, jnp.inf, dist)
        nb = jnp.dot(sel.astype(jnp.float32), keys,
                     preferred_element_type=jnp.float32,
                     precision=jax.lax.Precision.HIGHEST)
        o_ref[0, :, j, :] = nb


def kernel(inputs):
    B, N, F = inputs.shape
    inputs_t = jnp.transpose(inputs, (0, 2, 1))
    grid = (B, N // Q)
    out = pl.pallas_call(
        _knn_body,
        grid=grid,
        in_specs=[
            pl.BlockSpec((1, Q, F), lambda b, nb: (b, nb, 0)),
            pl.BlockSpec((1, F, N), lambda b, nb: (b, 0, 0)),
            pl.BlockSpec((1, N, F), lambda b, nb: (b, 0, 0)),
        ],
        out_specs=pl.BlockSpec((1, Q, K + 1, F), lambda b, nb: (b, nb, 0, 0)),
        out_shape=jax.ShapeDtypeStruct((B, N, K + 1, F), jnp.float32),
        compiler_params=pltpu.CompilerParams(
            dimension_semantics=("parallel", "arbitrary"),
        ),
    )(inputs, inputs_t, inputs)
    return jnp.transpose(out, (0, 1, 3, 2))


# speedup vs baseline: 4.3187x; 3.1213x over previous
import functools

import jax
import jax.numpy as jnp
from jax.experimental import pallas as pl
from jax.experimental.pallas import tpu as pltpu

K = 16
Q = 32


def _knn_body(q_ref, kt_ref, k_ref, o_ref):
    keys_t = kt_ref[0]
    keys = k_ref[0]
    n = keys.shape[0]
    queries_t = q_ref[0].T

    rows = []
    for q in range(Q):
        dq = jnp.abs(queries_t[:, q:q + 1] - keys_t)
        rows.append(jnp.sum(dq, axis=0, keepdims=True))
    dist = jnp.concatenate(rows, axis=0)

    col = jax.lax.broadcasted_iota(jnp.int32, dist.shape, 1)

    for j in range(K + 1):
        mn = jnp.min(dist, axis=1, keepdims=True)
        eq = dist == mn
        idxv = jnp.min(jnp.where(eq, col, n), axis=1)
        sel = col == idxv[:, None]
        dist = jnp.where(sel, jnp.inf, dist)
        nb = jnp.dot(sel.astype(jnp.float32), keys,
                     preferred_element_type=jnp.float32,
                     precision=jax.lax.Precision.HIGHEST)
        o_ref[0, :, j, :] = nb


def kernel(inputs):
    B, N, F = inputs.shape
    inputs_t = jnp.transpose(inputs, (0, 2, 1))
    grid = (B, N // Q)
    out = pl.pallas_call(
        _knn_body,
        grid=grid,
        in_specs=[
            pl.BlockSpec((1, Q, F), lambda b, nb: (b, nb, 0)),
            pl.BlockSpec((1, F, N), lambda b, nb: (b, 0, 0)),
            pl.BlockSpec((1, N, F), lambda b, nb: (b, 0, 0)),
        ],
        out_specs=pl.BlockSpec((1, Q, K + 1, F), lambda b, nb: (b, nb, 0, 0)),
        out_shape=jax.ShapeDtypeStruct((B, N, K + 1, F), jnp.float32),
        compiler_params=pltpu.CompilerParams(
            dimension_semantics=("parallel", "arbitrary"),
        ),
    )(inputs, inputs_t, inputs)
    return jnp.transpose(out, (0, 1, 3, 2))


# speedup vs baseline: 6.4673x; 1.4975x over previous
import functools

import jax
import jax.numpy as jnp
from jax.experimental import pallas as pl
from jax.experimental.pallas import tpu as pltpu

K = 16
Q = 64


def _knn_body(q_ref, kt_ref, k_ref, o_ref):
    keys_t = kt_ref[0]
    keys = k_ref[0]
    n = keys.shape[0]
    queries_t = q_ref[0].T

    rows = []
    for q in range(Q):
        dq = jnp.abs(queries_t[:, q:q + 1] - keys_t)
        rows.append(jnp.sum(dq, axis=0, keepdims=True))
    dist = jnp.concatenate(rows, axis=0)

    col = jax.lax.broadcasted_iota(jnp.int32, dist.shape, 1)

    for j in range(K + 1):
        mn = jnp.min(dist, axis=1, keepdims=True)
        eq = dist == mn
        idxv = jnp.min(jnp.where(eq, col, n), axis=1)
        sel = col == idxv[:, None]
        dist = jnp.where(sel, jnp.inf, dist)
        nb = jnp.dot(sel.astype(jnp.float32), keys,
                     preferred_element_type=jnp.float32,
                     precision=jax.lax.Precision.HIGHEST)
        o_ref[0, :, j, :] = nb


def kernel(inputs):
    B, N, F = inputs.shape
    inputs_t = jnp.transpose(inputs, (0, 2, 1))
    grid = (B, N // Q)
    out = pl.pallas_call(
        _knn_body,
        grid=grid,
        in_specs=[
            pl.BlockSpec((1, Q, F), lambda b, nb: (b, nb, 0)),
            pl.BlockSpec((1, F, N), lambda b, nb: (b, 0, 0)),
            pl.BlockSpec((1, N, F), lambda b, nb: (b, 0, 0)),
        ],
        out_specs=pl.BlockSpec((1, Q, K + 1, F), lambda b, nb: (b, nb, 0, 0)),
        out_shape=jax.ShapeDtypeStruct((B, N, K + 1, F), jnp.float32),
        compiler_params=pltpu.CompilerParams(
            dimension_semantics=("parallel", "arbitrary"),
        ),
    )(inputs, inputs_t, inputs)
    return jnp.transpose(out, (0, 1, 3, 2))


# speedup vs baseline: 8.9502x; 1.3839x over previous
import functools

import jax
import jax.numpy as jnp
from jax.experimental import pallas as pl
from jax.experimental.pallas import tpu as pltpu

K = 16
Q = 128


def _knn_body(q_ref, kt_ref, k_ref, o_ref):
    keys_t = kt_ref[0]
    keys = k_ref[0]
    n = keys.shape[0]
    queries_t = q_ref[0].T

    rows = []
    for q in range(Q):
        dq = jnp.abs(queries_t[:, q:q + 1] - keys_t)
        rows.append(jnp.sum(dq, axis=0, keepdims=True))
    dist = jnp.concatenate(rows, axis=0)

    col = jax.lax.broadcasted_iota(jnp.int32, dist.shape, 1)

    for j in range(K + 1):
        mn = jnp.min(dist, axis=1, keepdims=True)
        eq = dist == mn
        idxv = jnp.min(jnp.where(eq, col, n), axis=1)
        sel = col == idxv[:, None]
        dist = jnp.where(sel, jnp.inf, dist)
        nb = jnp.dot(sel.astype(jnp.float32), keys,
                     preferred_element_type=jnp.float32,
                     precision=jax.lax.Precision.HIGHEST)
        o_ref[0, :, j, :] = nb


def kernel(inputs):
    B, N, F = inputs.shape
    inputs_t = jnp.transpose(inputs, (0, 2, 1))
    grid = (B, N // Q)
    out = pl.pallas_call(
        _knn_body,
        grid=grid,
        in_specs=[
            pl.BlockSpec((1, Q, F), lambda b, nb: (b, nb, 0)),
            pl.BlockSpec((1, F, N), lambda b, nb: (b, 0, 0)),
            pl.BlockSpec((1, N, F), lambda b, nb: (b, 0, 0)),
        ],
        out_specs=pl.BlockSpec((1, Q, K + 1, F), lambda b, nb: (b, nb, 0, 0)),
        out_shape=jax.ShapeDtypeStruct((B, N, K + 1, F), jnp.float32),
        compiler_params=pltpu.CompilerParams(
            dimension_semantics=("parallel", "arbitrary"),
        ),
    )(inputs, inputs_t, inputs)
    return jnp.transpose(out, (0, 1, 3, 2))


# speedup vs baseline: 9.4311x; 1.0537x over previous
import functools

import jax
import jax.numpy as jnp
from jax.experimental import pallas as pl
from jax.experimental.pallas import tpu as pltpu

K = 16
Q = 256


def _knn_body(q_ref, kt_ref, k_ref, o_ref):
    keys_t = kt_ref[0]
    keys = k_ref[0]
    n = keys.shape[0]
    queries_t = q_ref[0].T

    rows = []
    for q in range(Q):
        dq = jnp.abs(queries_t[:, q:q + 1] - keys_t)
        rows.append(jnp.sum(dq, axis=0, keepdims=True))
    dist = jnp.concatenate(rows, axis=0)

    col = jax.lax.broadcasted_iota(jnp.int32, dist.shape, 1)

    for j in range(K + 1):
        mn = jnp.min(dist, axis=1, keepdims=True)
        eq = dist == mn
        idxv = jnp.min(jnp.where(eq, col, n), axis=1)
        sel = col == idxv[:, None]
        dist = jnp.where(sel, jnp.inf, dist)
        nb = jnp.dot(sel.astype(jnp.float32), keys,
                     preferred_element_type=jnp.float32,
                     precision=jax.lax.Precision.HIGHEST)
        o_ref[0, :, j, :] = nb


def kernel(inputs):
    B, N, F = inputs.shape
    inputs_t = jnp.transpose(inputs, (0, 2, 1))
    grid = (B, N // Q)
    out = pl.pallas_call(
        _knn_body,
        grid=grid,
        in_specs=[
            pl.BlockSpec((1, Q, F), lambda b, nb: (b, nb, 0)),
            pl.BlockSpec((1, F, N), lambda b, nb: (b, 0, 0)),
            pl.BlockSpec((1, N, F), lambda b, nb: (b, 0, 0)),
        ],
        out_specs=pl.BlockSpec((1, Q, K + 1, F), lambda b, nb: (b, nb, 0, 0)),
        out_shape=jax.ShapeDtypeStruct((B, N, K + 1, F), jnp.float32),
        compiler_params=pltpu.CompilerParams(
            dimension_semantics=("parallel", "arbitrary"),
        ),
    )(inputs, inputs_t, inputs)
    return jnp.transpose(out, (0, 1, 3, 2))
